# baseline (speedup 1.0000x reference)
import functools

import jax
import jax.numpy as jnp
from jax import lax
from jax.experimental import pallas as pl
from jax.experimental.pallas import tpu as pltpu
from jax.experimental.pallas import tpu_sc as plsc

ROWS = 128
COLS = 100000


_K1 = 42
_K2 = 0x1BD11BDA ^ 42
_ROT = ((13, 15, 26, 6), (17, 29, 16, 24), (13, 15, 26, 6),
        (17, 29, 16, 24), (13, 15, 26, 6))
_KA = (_K1, _K2, None, _K1, _K2)
_KB = (_K2 + 1, 0 + 2, _K1 + 3, _K2 + 4, 0 + 5)


def _threefry_xor(x1):
    x0 = x1
    for g in range(5):
        first = g == 0
        for r in _ROT[g]:
            if first:
                first = False
            else:
                x0 = x0 + x1
            x1 = (x1 << r) | (x1 >> (32 - r))
            x1 = x1 ^ x0
        if _KA[g] is not None:
            x0 = x0 + jnp.uint32(_KA[g])
        x1 = x1 + jnp.uint32(_KB[g])
    return x0 ^ x1


def _bits_to_u(bits):
    f = lax.bitcast_convert_type((bits >> 9) | jnp.uint32(0x3F800000),
                                 jnp.float32) - jnp.float32(1.0)
    return jnp.maximum(jnp.float32(1e-12), f + jnp.float32(1e-12))


CS = 30720
SC_CH = 3840
SC_NCH = 4
RPW = 8
SC_UNROLL = 4
SC_HALF = CS // 2


def _ln(x):
    bits = lax.bitcast_convert_type(x, jnp.uint32)
    e = (bits >> 23).astype(jnp.int32) - 127
    m = lax.bitcast_convert_type((bits & jnp.uint32(0x007FFFFF))
                                 | jnp.uint32(0x3F800000), jnp.float32)
    big = m > jnp.float32(1.4142135)
    m = jnp.where(big, m * jnp.float32(0.5), m)
    e = jnp.where(big, e + 1, e).astype(jnp.float32)
    t = m - jnp.float32(1.0)
    s = t / (t + jnp.float32(2.0))
    z = s * s
    p = jnp.float32(2.0 / 9.0)
    p = p * z + jnp.float32(2.0 / 7.0)
    p = p * z + jnp.float32(2.0 / 5.0)
    p = p * z + jnp.float32(2.0 / 3.0)
    lnm = (s * z) * p + (s + s)
    return e * jnp.float32(0.6931471805599453) + lnm


def _r16(x1_u32, lp_vec):
    return (-_ln(_bits_to_u(_threefry_xor(x1_u32)))) * jnp.exp(-lp_vec)


@functools.cache
def _make_sc():
    mesh = plsc.VectorSubcoreMesh(core_axis_name="c", subcore_axis_name="s")

    @functools.partial(
        pl.kernel, mesh=mesh,
        out_type=(jax.ShapeDtypeStruct((32, RPW, 16), jnp.int32),
                  jax.ShapeDtypeStruct((32, RPW, 16), jnp.float32)),
        scratch_types=[
            pltpu.VMEM((RPW, SC_CH), jnp.float32),
            pltpu.VMEM((RPW, SC_CH), jnp.float32),
            pltpu.VMEM((RPW, 16), jnp.float32),
            pltpu.VMEM((RPW, 16), jnp.int32),
            pltpu.SemaphoreType.DMA,
            pltpu.SemaphoreType.DMA,
        ],
    )
    def sc_kernel(lp_hbm, outi_hbm, outv_hbm, buf_a, buf_b, bvst, bist,
                  sem_a, sem_b):
        wid = lax.axis_index("s") * 2 + lax.axis_index("c")
        half = wid // 16
        grp = wid % 16
        r0 = grp * RPW
        cbase = half * SC_HALF
        lane = lax.iota(jnp.int32, 16)
        pos_inf = jnp.full((16,), jnp.inf, jnp.float32)

        for rr in range(RPW):
            bvst[rr, :] = pos_inf
            bist[rr, :] = jnp.zeros((16,), jnp.int32)

        def start(c, buf, sem):
            pltpu.async_copy(
                lp_hbm.at[pl.ds(r0, RPW), pl.ds(cbase + c * SC_CH, SC_CH)],
                buf, sem)

        def wait(buf, sem):
            pltpu.make_async_copy(
                lp_hbm.at[pl.ds(0, RPW), pl.ds(0, SC_CH)], buf, sem).wait()

        def compute(c0, buf):
            nit = SC_CH // (16 * SC_UNROLL)

            def row_body(rr, _):
                row = r0 + rr
                base_u = ((row * COLS + c0 + _K1).astype(jnp.uint32)
                          + lane.astype(jnp.uint32))
                base_i = c0 + lane

                def body(k, carry):
                    bv, bi = carry
                    for uu in range(SC_UNROLL):
                        off = k * (16 * SC_UNROLL) + uu * 16
                        lpv = buf[rr, pl.ds(off, 16)]
                        s = _r16(base_u + off.astype(jnp.uint32), lpv)
                        m = s < bv
                        bv = jnp.where(m, s, bv)
                        bi = jnp.where(m, base_i + off, bi)
                    return bv, bi

                bv, bi = lax.fori_loop(0, nit, body,
                                       (bvst[rr, :], bist[rr, :]))
                bvst[rr, :] = bv
                bist[rr, :] = bi
                return 0

            lax.fori_loop(0, RPW, row_body, 0)

        start(0, buf_a, sem_a)
        start(1, buf_b, sem_b)

        def pair(cc, _):
            c = cc * 2
            wait(buf_a, sem_a)
            compute(cbase + c * SC_CH, buf_a)

            @pl.when(c + 2 < SC_NCH)
            def _():
                start(c + 2, buf_a, sem_a)

            wait(buf_b, sem_b)
            compute(cbase + (c + 1) * SC_CH, buf_b)

            @pl.when(c + 3 < SC_NCH)
            def _():
                start(c + 3, buf_b, sem_b)

            return 0

        lax.fori_loop(0, SC_NCH // 2, pair, 0)

        pltpu.sync_copy(bist, outi_hbm.at[wid])
        pltpu.sync_copy(bvst, outv_hbm.at[wid])

    return sc_kernel


BC = 2048
SUB = 128
TC_BLK0 = CS // BC
NBLK = (COLS - CS + BC - 1) // BC


def _tc_body(lp_ref, outv_ref, outi_ref, bv_ref, bi_ref):
    j = pl.program_id(0)

    @pl.when(j == 0)
    def _init():
        bv_ref[...] = jnp.full((ROWS, SUB), -jnp.inf, jnp.float32)
        bi_ref[...] = jnp.zeros((ROWS, SUB), jnp.int32)

    rowc = jax.lax.broadcasted_iota(jnp.int32, (ROWS, SUB), 0) * COLS
    ci = jax.lax.broadcasted_iota(jnp.int32, (ROWS, SUB), 1)
    base = rowc + ci

    bv = bv_ref[...]
    bi = bi_ref[...]
    for k in range(BC // SUB):
        lp = lp_ref[:, k * SUB:(k + 1) * SUB]
        c0 = CS + j * BC + k * SUB
        bits = _threefry_xor((base + (c0 + _K1)).astype(jnp.uint32))
        u = _bits_to_u(bits)
        score = lp - jnp.log(-jnp.log(u))
        score = jnp.where(ci + c0 < COLS, score, -jnp.inf)
        m = score > bv
        bv = jnp.where(m, score, bv)
        bi = jnp.where(m, ci + c0, bi)
    bv_ref[...] = bv
    bi_ref[...] = bi

    @pl.when(j == NBLK - 1)
    def _final():
        outv_ref[...] = bv_ref[...]
        outi_ref[...] = bi_ref[...]


def _tc_part(log_p):
    return pl.pallas_call(
        _tc_body,
        grid=(NBLK,),
        in_specs=[pl.BlockSpec((ROWS, BC), lambda j: (0, TC_BLK0 + j))],
        out_specs=(pl.BlockSpec((ROWS, SUB), lambda j: (0, 0)),
                   pl.BlockSpec((ROWS, SUB), lambda j: (0, 0))),
        out_shape=(jax.ShapeDtypeStruct((ROWS, SUB), jnp.float32),
                   jax.ShapeDtypeStruct((ROWS, SUB), jnp.int32)),
        scratch_shapes=[
            pltpu.VMEM((ROWS, SUB), jnp.float32),
            pltpu.VMEM((ROWS, SUB), jnp.int32),
        ],
    )(log_p)


@jax.jit
def kernel(log_p):
    sci, scv = _make_sc()(log_p)
    tcv, tci = _tc_part(log_p)
    scv = -jnp.log(scv)
    scv_r = jnp.concatenate([scv[:16].reshape(ROWS, 16),
                             scv[16:].reshape(ROWS, 16)], axis=1)
    sci_r = jnp.concatenate([sci[:16].reshape(ROWS, 16),
                             sci[16:].reshape(ROWS, 16)], axis=1)
    v = jnp.concatenate([scv_r, tcv], axis=1)
    i = jnp.concatenate([sci_r, tci], axis=1)
    mx = jnp.max(v, axis=1, keepdims=True)
    cand = jnp.where(v == mx, i, jnp.int32(COLS + 1))
    return jnp.min(cand, axis=1)


# speedup vs baseline: 1.2316x; 1.2316x over previous
import functools

import jax
import jax.numpy as jnp
from jax import lax
from jax.experimental import pallas as pl
from jax.experimental.pallas import tpu as pltpu
from jax.experimental.pallas import tpu_sc as plsc

ROWS = 128
COLS = 100000


_K1 = 42
_K2 = 0x1BD11BDA ^ 42
_ROT = ((13, 15, 26, 6), (17, 29, 16, 24), (13, 15, 26, 6),
        (17, 29, 16, 24), (13, 15, 26, 6))
_KA = (_K1, _K2, None, _K1, _K2)
_KB = (_K2 + 1, 0 + 2, _K1 + 3, _K2 + 4, 0 + 5)


def _threefry_xor(x1):
    x0 = x1
    for g in range(5):
        first = g == 0
        for r in _ROT[g]:
            if first:
                first = False
            else:
                x0 = x0 + x1
            x1 = (x1 << r) | (x1 >> (32 - r))
            x1 = x1 ^ x0
        if _KA[g] is not None:
            x0 = x0 + jnp.uint32(_KA[g])
        x1 = x1 + jnp.uint32(_KB[g])
    return x0 ^ x1


def _bits_to_u(bits):
    f = lax.bitcast_convert_type((bits >> 9) | jnp.uint32(0x3F800000),
                                 jnp.float32) - jnp.float32(1.0)
    return jnp.maximum(jnp.float32(1e-12), f + jnp.float32(1e-12))


CS = 20480
SC_CH = 5120
SC_NCH = 2
RPW = 8
SC_UNROLL = 4
SC_HALF = CS // 2


def _ln(x):
    bits = lax.bitcast_convert_type(x, jnp.uint32)
    e = (bits >> 23).astype(jnp.int32) - 127
    m = lax.bitcast_convert_type((bits & jnp.uint32(0x007FFFFF))
                                 | jnp.uint32(0x3F800000), jnp.float32)
    big = m > jnp.float32(1.4142135)
    m = jnp.where(big, m * jnp.float32(0.5), m)
    e = jnp.where(big, e + 1, e).astype(jnp.float32)
    t = m - jnp.float32(1.0)
    s = t / (t + jnp.float32(2.0))
    z = s * s
    p = jnp.float32(2.0 / 9.0)
    p = p * z + jnp.float32(2.0 / 7.0)
    p = p * z + jnp.float32(2.0 / 5.0)
    p = p * z + jnp.float32(2.0 / 3.0)
    lnm = (s * z) * p + (s + s)
    return e * jnp.float32(0.6931471805599453) + lnm


def _r16(x1_u32, lp_vec):
    return (-_ln(_bits_to_u(_threefry_xor(x1_u32)))) * jnp.exp(-lp_vec)


@functools.cache
def _make_sc():
    mesh = plsc.VectorSubcoreMesh(core_axis_name="c", subcore_axis_name="s")

    @functools.partial(
        pl.kernel, mesh=mesh,
        out_type=(jax.ShapeDtypeStruct((32, RPW, 16), jnp.int32),
                  jax.ShapeDtypeStruct((32, RPW, 16), jnp.float32)),
        scratch_types=[
            pltpu.VMEM((RPW, SC_CH), jnp.float32),
            pltpu.VMEM((RPW, SC_CH), jnp.float32),
            pltpu.VMEM((RPW, 16), jnp.float32),
            pltpu.VMEM((RPW, 16), jnp.int32),
            pltpu.SemaphoreType.DMA,
            pltpu.SemaphoreType.DMA,
        ],
    )
    def sc_kernel(lp_hbm, outi_hbm, outv_hbm, buf_a, buf_b, bvst, bist,
                  sem_a, sem_b):
        wid = lax.axis_index("s") * 2 + lax.axis_index("c")
        half = wid // 16
        grp = wid % 16
        r0 = grp * RPW
        cbase = half * SC_HALF
        lane = lax.iota(jnp.int32, 16)
        pos_inf = jnp.full((16,), jnp.inf, jnp.float32)

        for rr in range(RPW):
            bvst[rr, :] = pos_inf
            bist[rr, :] = jnp.zeros((16,), jnp.int32)

        def start(c, buf, sem):
            pltpu.async_copy(
                lp_hbm.at[pl.ds(r0, RPW), pl.ds(cbase + c * SC_CH, SC_CH)],
                buf, sem)

        def wait(buf, sem):
            pltpu.make_async_copy(
                lp_hbm.at[pl.ds(0, RPW), pl.ds(0, SC_CH)], buf, sem).wait()

        def compute(c0, buf):
            nit = SC_CH // (16 * SC_UNROLL)

            def row_body(rr, _):
                row = r0 + rr
                base_u = ((row * COLS + c0 + _K1).astype(jnp.uint32)
                          + lane.astype(jnp.uint32))
                base_i = c0 + lane

                def body(k, carry):
                    bv, bi = carry
                    for uu in range(SC_UNROLL):
                        off = k * (16 * SC_UNROLL) + uu * 16
                        lpv = buf[rr, pl.ds(off, 16)]
                        s = _r16(base_u + off.astype(jnp.uint32), lpv)
                        m = s < bv
                        bv = jnp.where(m, s, bv)
                        bi = jnp.where(m, base_i + off, bi)
                    return bv, bi

                bv, bi = lax.fori_loop(0, nit, body,
                                       (bvst[rr, :], bist[rr, :]))
                bvst[rr, :] = bv
                bist[rr, :] = bi
                return 0

            lax.fori_loop(0, RPW, row_body, 0)

        start(0, buf_a, sem_a)
        start(1, buf_b, sem_b)

        def pair(cc, _):
            c = cc * 2
            wait(buf_a, sem_a)
            compute(cbase + c * SC_CH, buf_a)

            @pl.when(c + 2 < SC_NCH)
            def _():
                start(c + 2, buf_a, sem_a)

            wait(buf_b, sem_b)
            compute(cbase + (c + 1) * SC_CH, buf_b)

            @pl.when(c + 3 < SC_NCH)
            def _():
                start(c + 3, buf_b, sem_b)

            return 0

        lax.fori_loop(0, SC_NCH // 2, pair, 0)

        pltpu.sync_copy(bist, outi_hbm.at[wid])
        pltpu.sync_copy(bvst, outv_hbm.at[wid])

    return sc_kernel


BC = 2048
SUB = 128
TC_BLK0 = CS // BC
NBLK = (COLS - CS + BC - 1) // BC


def _tc_body(lp_ref, outv_ref, outi_ref, bv_ref, bi_ref):
    j = pl.program_id(0)

    @pl.when(j == 0)
    def _init():
        bv_ref[...] = jnp.full((ROWS, SUB), -jnp.inf, jnp.float32)
        bi_ref[...] = jnp.zeros((ROWS, SUB), jnp.int32)

    rowc = jax.lax.broadcasted_iota(jnp.int32, (ROWS, SUB), 0) * COLS
    ci = jax.lax.broadcasted_iota(jnp.int32, (ROWS, SUB), 1)
    base = rowc + ci

    for k in range(BC // SUB):
        lp = lp_ref[:, k * SUB:(k + 1) * SUB]
        c0 = CS + j * BC + k * SUB
        bits = _threefry_xor((base + (c0 + _K1)).astype(jnp.uint32))
        u = _bits_to_u(bits)
        score = lp - jnp.log(-jnp.log(u))
        score = jnp.where(ci + c0 < COLS, score, -jnp.inf)
        bv = bv_ref[...]
        m = score > bv
        bv_ref[...] = jnp.where(m, score, bv)
        bi_ref[...] = jnp.where(m, ci + c0, bi_ref[...])

    @pl.when(j == NBLK - 1)
    def _final():
        outv_ref[...] = bv_ref[...]
        outi_ref[...] = bi_ref[...]


def _tc_part(log_p):
    return pl.pallas_call(
        _tc_body,
        grid=(NBLK,),
        in_specs=[pl.BlockSpec((ROWS, BC), lambda j: (0, TC_BLK0 + j))],
        out_specs=(pl.BlockSpec((ROWS, SUB), lambda j: (0, 0)),
                   pl.BlockSpec((ROWS, SUB), lambda j: (0, 0))),
        out_shape=(jax.ShapeDtypeStruct((ROWS, SUB), jnp.float32),
                   jax.ShapeDtypeStruct((ROWS, SUB), jnp.int32)),
        scratch_shapes=[
            pltpu.VMEM((ROWS, SUB), jnp.float32),
            pltpu.VMEM((ROWS, SUB), jnp.int32),
        ],
    )(log_p)


@jax.jit
def kernel(log_p):
    sci, scv = _make_sc()(log_p)
    tcv, tci = _tc_part(log_p)
    scv = -jnp.log(scv)
    scv_r = jnp.concatenate([scv[:16].reshape(ROWS, 16),
                             scv[16:].reshape(ROWS, 16)], axis=1)
    sci_r = jnp.concatenate([sci[:16].reshape(ROWS, 16),
                             sci[16:].reshape(ROWS, 16)], axis=1)
    v = jnp.concatenate([scv_r, tcv], axis=1)
    i = jnp.concatenate([sci_r, tci], axis=1)
    mx = jnp.max(v, axis=1, keepdims=True)
    cand = jnp.where(v == mx, i, jnp.int32(COLS + 1))
    return jnp.min(cand, axis=1)


# speedup vs baseline: 1.2328x; 1.0010x over previous
import functools

import jax
import jax.numpy as jnp
from jax import lax
from jax.experimental import pallas as pl
from jax.experimental.pallas import tpu as pltpu
from jax.experimental.pallas import tpu_sc as plsc

ROWS = 128
COLS = 100000


_K1 = 42
_K2 = 0x1BD11BDA ^ 42
_ROT = ((13, 15, 26, 6), (17, 29, 16, 24), (13, 15, 26, 6),
        (17, 29, 16, 24), (13, 15, 26, 6))
_KA = (_K1, _K2, None, _K1, _K2)
_KB = (_K2 + 1, 0 + 2, _K1 + 3, _K2 + 4, 0 + 5)


def _threefry_xor(x1):
    x0 = x1
    for g in range(5):
        first = g == 0
        for r in _ROT[g]:
            if first:
                first = False
            else:
                x0 = x0 + x1
            x1 = (x1 << r) | (x1 >> (32 - r))
            x1 = x1 ^ x0
        if _KA[g] is not None:
            x0 = x0 + jnp.uint32(_KA[g])
        x1 = x1 + jnp.uint32(_KB[g])
    return x0 ^ x1


def _bits_to_u(bits):
    return lax.bitcast_convert_type((bits >> 9) | jnp.uint32(0x3F800000),
                                    jnp.float32) - jnp.float32(1.0)


CS = 23040
SC_CH = 5760
SC_NCH = 2
RPW = 8
SC_UNROLL = 4
SC_HALF = CS // 2


def _ln(x):
    bits = lax.bitcast_convert_type(x, jnp.uint32)
    e = (bits >> 23).astype(jnp.int32) - 127
    m = lax.bitcast_convert_type((bits & jnp.uint32(0x007FFFFF))
                                 | jnp.uint32(0x3F800000), jnp.float32)
    big = m > jnp.float32(1.4142135)
    m = jnp.where(big, m * jnp.float32(0.5), m)
    e = jnp.where(big, e + 1, e).astype(jnp.float32)
    t = m - jnp.float32(1.0)
    s = t / (t + jnp.float32(2.0))
    z = s * s
    p = jnp.float32(2.0 / 9.0)
    p = p * z + jnp.float32(2.0 / 7.0)
    p = p * z + jnp.float32(2.0 / 5.0)
    p = p * z + jnp.float32(2.0 / 3.0)
    lnm = (s * z) * p + (s + s)
    return e * jnp.float32(0.6931471805599453) + lnm


def _r16(x1_u32, lp_vec):
    return (-_ln(_bits_to_u(_threefry_xor(x1_u32)))) * jnp.exp(-lp_vec)


@functools.cache
def _make_sc():
    mesh = plsc.VectorSubcoreMesh(core_axis_name="c", subcore_axis_name="s")

    @functools.partial(
        pl.kernel, mesh=mesh,
        out_type=(jax.ShapeDtypeStruct((32, RPW, 16), jnp.int32),
                  jax.ShapeDtypeStruct((32, RPW, 16), jnp.float32)),
        scratch_types=[
            pltpu.VMEM((RPW, SC_CH), jnp.float32),
            pltpu.VMEM((RPW, SC_CH), jnp.float32),
            pltpu.VMEM((RPW, 16), jnp.float32),
            pltpu.VMEM((RPW, 16), jnp.int32),
            pltpu.SemaphoreType.DMA,
            pltpu.SemaphoreType.DMA,
        ],
    )
    def sc_kernel(lp_hbm, outi_hbm, outv_hbm, buf_a, buf_b, bvst, bist,
                  sem_a, sem_b):
        wid = lax.axis_index("s") * 2 + lax.axis_index("c")
        half = wid // 16
        grp = wid % 16
        r0 = grp * RPW
        cbase = half * SC_HALF
        lane = lax.iota(jnp.int32, 16)
        pos_inf = jnp.full((16,), jnp.inf, jnp.float32)

        for rr in range(RPW):
            bvst[rr, :] = pos_inf
            bist[rr, :] = jnp.zeros((16,), jnp.int32)

        def start(c, buf, sem):
            pltpu.async_copy(
                lp_hbm.at[pl.ds(r0, RPW), pl.ds(cbase + c * SC_CH, SC_CH)],
                buf, sem)

        def wait(buf, sem):
            pltpu.make_async_copy(
                lp_hbm.at[pl.ds(0, RPW), pl.ds(0, SC_CH)], buf, sem).wait()

        def compute(c0, buf):
            nit = SC_CH // (16 * SC_UNROLL)

            def row_body(rr, _):
                row = r0 + rr
                base_u = ((row * COLS + c0 + _K1).astype(jnp.uint32)
                          + lane.astype(jnp.uint32))
                base_i = c0 + lane

                def body(k, carry):
                    bv, bi = carry
                    for uu in range(SC_UNROLL):
                        off = k * (16 * SC_UNROLL) + uu * 16
                        lpv = buf[rr, pl.ds(off, 16)]
                        s = _r16(base_u + off.astype(jnp.uint32), lpv)
                        m = s < bv
                        bv = jnp.where(m, s, bv)
                        bi = jnp.where(m, base_i + off, bi)
                    return bv, bi

                bv, bi = lax.fori_loop(0, nit, body,
                                       (bvst[rr, :], bist[rr, :]))
                bvst[rr, :] = bv
                bist[rr, :] = bi
                return 0

            lax.fori_loop(0, RPW, row_body, 0)

        start(0, buf_a, sem_a)
        start(1, buf_b, sem_b)

        def pair(cc, _):
            c = cc * 2
            wait(buf_a, sem_a)
            compute(cbase + c * SC_CH, buf_a)

            @pl.when(c + 2 < SC_NCH)
            def _():
                start(c + 2, buf_a, sem_a)

            wait(buf_b, sem_b)
            compute(cbase + (c + 1) * SC_CH, buf_b)

            @pl.when(c + 3 < SC_NCH)
            def _():
                start(c + 3, buf_b, sem_b)

            return 0

        lax.fori_loop(0, SC_NCH // 2, pair, 0)

        for rr in range(RPW):
            bvst[rr, :] = jnp.float32(0.0) - _ln(bvst[rr, :])

        pltpu.sync_copy(bist, outi_hbm.at[wid])
        pltpu.sync_copy(bvst, outv_hbm.at[wid])

    return sc_kernel


BC = 1920
SUB = 128
TC_BLK0 = CS // BC
NBLK = (COLS - CS + BC - 1) // BC


def _tc_body(lp_ref, outv_ref, outi_ref, bv_ref, bi_ref):
    j = pl.program_id(0)

    @pl.when(j == 0)
    def _init():
        bv_ref[...] = jnp.full((ROWS, SUB), -jnp.inf, jnp.float32)
        bi_ref[...] = jnp.zeros((ROWS, SUB), jnp.int32)

    rowc = jax.lax.broadcasted_iota(jnp.int32, (ROWS, SUB), 0) * COLS
    ci = jax.lax.broadcasted_iota(jnp.int32, (ROWS, SUB), 1)
    base = rowc + ci

    for k in range(BC // SUB):
        lp = lp_ref[:, k * SUB:(k + 1) * SUB]
        c0 = CS + j * BC + k * SUB
        bits = _threefry_xor((base + (c0 + _K1)).astype(jnp.uint32))
        u = _bits_to_u(bits)
        score = lp - jnp.log(-jnp.log(u))
        score = jnp.where(ci + c0 < COLS, score, -jnp.inf)
        bv = bv_ref[...]
        m = score > bv
        bv_ref[...] = jnp.where(m, score, bv)
        bi_ref[...] = jnp.where(m, ci + c0, bi_ref[...])

    @pl.when(j == NBLK - 1)
    def _final():
        outv_ref[...] = bv_ref[...]
        outi_ref[...] = bi_ref[...]


def _tc_part(log_p):
    return pl.pallas_call(
        _tc_body,
        grid=(NBLK,),
        in_specs=[pl.BlockSpec((ROWS, BC), lambda j: (0, TC_BLK0 + j))],
        out_specs=(pl.BlockSpec((ROWS, SUB), lambda j: (0, 0)),
                   pl.BlockSpec((ROWS, SUB), lambda j: (0, 0))),
        out_shape=(jax.ShapeDtypeStruct((ROWS, SUB), jnp.float32),
                   jax.ShapeDtypeStruct((ROWS, SUB), jnp.int32)),
        scratch_shapes=[
            pltpu.VMEM((ROWS, SUB), jnp.float32),
            pltpu.VMEM((ROWS, SUB), jnp.int32),
        ],
    )(log_p)


@jax.jit
def kernel(log_p):
    sci, scv = _make_sc()(log_p)
    tcv, tci = _tc_part(log_p)
    scv_r = jnp.concatenate([scv[:16].reshape(ROWS, 16),
                             scv[16:].reshape(ROWS, 16)], axis=1)
    sci_r = jnp.concatenate([sci[:16].reshape(ROWS, 16),
                             sci[16:].reshape(ROWS, 16)], axis=1)
    v = jnp.concatenate([scv_r, tcv], axis=1)
    i = jnp.concatenate([sci_r, tci], axis=1)
    mx = jnp.max(v, axis=1, keepdims=True)
    cand = jnp.where(v == mx, i, jnp.int32(COLS + 1))
    return jnp.min(cand, axis=1)
